# baseline (speedup 1.0000x reference)
import functools
import math

import jax
import jax.numpy as jnp
from jax import lax
from jax.experimental import pallas as pl
from jax.experimental.pallas import tpu as pltpu
from jax.experimental.pallas import tpu_sc as plsc

D = 64
SCALE = math.sqrt(D)
L = 16


def _make_kernel(V, B, C):
    info = plsc.get_sparse_core_info()
    NC, NS = info.num_cores, info.num_subcores
    NW = NC * NS
    assert B % NW == 0
    b_per_w = B // NW
    assert b_per_w % C == 0
    n_chunks = b_per_w // C

    mesh = plsc.VectorSubcoreMesh(core_axis_name="c", subcore_axis_name="s")

    @functools.partial(
        pl.kernel,
        mesh=mesh,
        out_type=jax.ShapeDtypeStruct((B, D), jnp.float32),
        scratch_types=[
            pltpu.VMEM((C,), jnp.int32),
            pltpu.VMEM((C,), jnp.int32),
            pltpu.VMEM((C, 2 * D), jnp.float32),
            pltpu.VMEM((C, D), jnp.float32),
            pltpu.SemaphoreType.DMA,
        ],
        compiler_params=pltpu.CompilerParams(needs_layout_passes=False),
    )
    def k(x_hbm, table2_hbm, out_hbm, idx_v, idx2_v, gath_v, outst_v, gsem):
        wid = lax.axis_index("s") * NC + lax.axis_index("c")
        base = wid * b_per_w
        iota = lax.iota(jnp.int32, L)

        def chunk_body(g, carry):
            off = base + g * C
            pltpu.sync_copy(x_hbm.at[pl.ds(off, C)], idx_v)

            def half_body(i, carry2):
                v = idx_v[pl.ds(i * L, L)]
                idx2_v[pl.ds(i * L, L)] = lax.shift_right_logical(v, 1)
                return carry2

            lax.fori_loop(0, C // L, half_body, 0)
            pltpu.async_copy(table2_hbm.at[idx2_v], gath_v, gsem).wait()

            def sel_body(grp, carry2):
                rows = grp * L + iota
                par = lax.bitwise_and(idx_v[pl.ds(grp * L, L)], 1)
                colbase = par * D
                for c in range(D):
                    colv = colbase + c
                    vals = plsc.load_gather(gath_v, [rows, colv])
                    plsc.store_scatter(outst_v,
                                       [rows, jnp.full((L,), c, jnp.int32)],
                                       vals * SCALE)
                return carry2

            lax.fori_loop(0, C // L, sel_body, 0)
            pltpu.sync_copy(outst_v, out_hbm.at[pl.ds(off, C)])
            return carry

        lax.fori_loop(0, n_chunks, chunk_body, 0)

    return k


def kernel(x, table):
    V = table.shape[0]
    B = x.shape[0] * x.shape[1]
    x_flat = x.reshape(B)
    table2 = table.reshape(V // 2, 2 * D)
    out = _make_kernel(V, B, 256)(x_flat, table2)
    return out.reshape(x.shape[0], x.shape[1], D)


# speedup vs baseline: 1.5333x; 1.5333x over previous
import functools
import math

import jax
import jax.numpy as jnp
from jax import lax
from jax.experimental import pallas as pl
from jax.experimental.pallas import tpu as pltpu
from jax.experimental.pallas import tpu_sc as plsc

D = 64
SCALE = math.sqrt(D)
L = 16


def _make_kernel(V, B, C):
    info = plsc.get_sparse_core_info()
    NC, NS = info.num_cores, info.num_subcores
    NW = NC * NS
    assert B % NW == 0
    b_per_w = B // NW
    assert b_per_w % C == 0
    n_chunks = b_per_w // C

    mesh = plsc.VectorSubcoreMesh(core_axis_name="c", subcore_axis_name="s")

    @functools.partial(
        pl.kernel,
        mesh=mesh,
        out_type=jax.ShapeDtypeStruct((B, D), jnp.float32),
        scratch_types=[
            pltpu.VMEM((C,), jnp.int32),
            pltpu.VMEM((C,), jnp.int32),
            pltpu.VMEM((C, 2 * D), jnp.float32),
            pltpu.VMEM((C, D), jnp.float32),
            pltpu.SemaphoreType.DMA,
        ],
        compiler_params=pltpu.CompilerParams(needs_layout_passes=False,
                                             disable_bounds_checks=True),
    )
    def k(x_hbm, table2_hbm, out_hbm, idx_v, idx2_v, gath_v, outst_v, gsem):
        wid = lax.axis_index("s") * NC + lax.axis_index("c")
        base = wid * b_per_w
        iota = lax.iota(jnp.int32, L)

        def chunk_body(g, carry):
            off = base + g * C
            pltpu.sync_copy(x_hbm.at[pl.ds(off, C)], idx_v)

            @plsc.parallel_loop(0, C // L)
            def half_body(i):
                v = idx_v[pl.ds(i * L, L)]
                idx2_v[pl.ds(i * L, L)] = lax.shift_right_logical(v, 1)

            pltpu.async_copy(table2_hbm.at[idx2_v], gath_v, gsem).wait()

            @plsc.parallel_loop(0, C // L)
            def sel_body(grp):
                rows = grp * L + iota
                par = lax.bitwise_and(idx_v[pl.ds(grp * L, L)], 1)
                colbase = par * D
                for c in range(D):
                    colv = colbase + c
                    vals = plsc.load_gather(gath_v, [rows, colv])
                    plsc.store_scatter(outst_v,
                                       [rows, jnp.full((L,), c, jnp.int32)],
                                       vals * SCALE)
            pltpu.sync_copy(outst_v, out_hbm.at[pl.ds(off, C)])
            return carry

        lax.fori_loop(0, n_chunks, chunk_body, 0)

    return k


def kernel(x, table):
    V = table.shape[0]
    B = x.shape[0] * x.shape[1]
    x_flat = x.reshape(B)
    table2 = table.reshape(V // 2, 2 * D)
    out = _make_kernel(V, B, 256)(x_flat, table2)
    return out.reshape(x.shape[0], x.shape[1], D)


# speedup vs baseline: 2.4852x; 1.6208x over previous
import functools
import math

import jax
import jax.numpy as jnp
from jax import lax
from jax.experimental import pallas as pl
from jax.experimental.pallas import tpu as pltpu
from jax.experimental.pallas import tpu_sc as plsc

VOCAB = 1000000
D = 64
SCALE = math.sqrt(D)
L = 16


def _make_kernel(B, C):
    info = plsc.get_sparse_core_info()
    NC, NS = info.num_cores, info.num_subcores
    NW = NC * NS
    assert B % NW == 0
    b_per_w = B // NW
    assert b_per_w % C == 0
    n_chunks = b_per_w // C

    mesh = plsc.VectorSubcoreMesh(core_axis_name="c", subcore_axis_name="s")

    @functools.partial(
        pl.kernel,
        mesh=mesh,
        out_type=jax.ShapeDtypeStruct((B, D), jnp.float32),
        scratch_types=[
            pltpu.VMEM((2, C), jnp.int32),
            pltpu.VMEM((2, C, D), jnp.float32),
            [pltpu.SemaphoreType.DMA] * 2,
            [pltpu.SemaphoreType.DMA] * 2,
        ],
        compiler_params=pltpu.CompilerParams(
            use_tc_tiling_on_sc=False, disable_bounds_checks=True),
    )
    def k(x_hbm, table_hbm, out_hbm, idx_v, rows_v, gsem, wsem):
        wid = lax.axis_index("s") * NC + lax.axis_index("c")
        base = wid * b_per_w

        def scale_chunk(b):
            @plsc.parallel_loop(0, C)
            def scale_body(r):
                for d in range(D // L):
                    sl = (b, r, pl.ds(d * L, L))
                    rows_v[sl] = rows_v[sl] * SCALE

        for g in range(n_chunks + 1):
            if g < n_chunks:
                b = g % 2
                off = base + g * C
                if g >= 2:
                    pltpu.make_async_copy(
                        rows_v.at[b], out_hbm.at[pl.ds(base + (g - 2) * C, C)],
                        wsem[b]).wait()
                pltpu.sync_copy(x_hbm.at[pl.ds(off, C)], idx_v.at[b])
                pltpu.async_copy(table_hbm.at[idx_v.at[b]], rows_v.at[b],
                                 gsem[b])
            if g >= 1:
                p = (g - 1) % 2
                poff = base + (g - 1) * C
                pltpu.make_async_copy(table_hbm.at[idx_v.at[p]],
                                      rows_v.at[p], gsem[p]).wait()
                scale_chunk(p)
                pltpu.async_copy(rows_v.at[p], out_hbm.at[pl.ds(poff, C)],
                                 wsem[p])

        for g in (n_chunks - 2, n_chunks - 1):
            b = g % 2
            pltpu.make_async_copy(
                rows_v.at[b], out_hbm.at[pl.ds(base + g * C, C)],
                wsem[b]).wait()

    return k


def kernel(x, table):
    B = x.shape[0] * x.shape[1]
    x_flat = x.reshape(B)
    out = _make_kernel(B, 800)(x_flat, table)
    return out.reshape(x.shape[0], x.shape[1], D)


# speedup vs baseline: 2.5299x; 1.0180x over previous
import functools
import math

import jax
import jax.numpy as jnp
from jax import lax
from jax.experimental import pallas as pl
from jax.experimental.pallas import tpu as pltpu
from jax.experimental.pallas import tpu_sc as plsc

VOCAB = 1000000
D = 64
SCALE = math.sqrt(D)
L = 16


def _make_kernel(B, C):
    info = plsc.get_sparse_core_info()
    NC, NS = info.num_cores, info.num_subcores
    NW = NC * NS
    assert B % NW == 0
    b_per_w = B // NW
    assert b_per_w % C == 0
    n_chunks = b_per_w // C

    mesh = plsc.VectorSubcoreMesh(core_axis_name="c", subcore_axis_name="s")

    @functools.partial(
        pl.kernel,
        mesh=mesh,
        out_type=jax.ShapeDtypeStruct((B, D), jnp.float32),
        scratch_types=[
            pltpu.VMEM((2, C), jnp.int32),
            pltpu.VMEM((2, C, D), jnp.float32),
            [pltpu.SemaphoreType.DMA] * 2,
            [pltpu.SemaphoreType.DMA] * 2,
        ],
        compiler_params=pltpu.CompilerParams(
            use_tc_tiling_on_sc=False, disable_bounds_checks=True),
    )
    def k(x_hbm, table_hbm, out_hbm, idx_v, rows_v, gsem, wsem):
        wid = lax.axis_index("s") * NC + lax.axis_index("c")
        base = wid * b_per_w

        def scale_chunk(b):
            @plsc.parallel_loop(0, C // 8)
            def scale_body(r):
                for rr in range(8):
                    for d in range(D // L):
                        sl = (b, r * 8 + rr, pl.ds(d * L, L))
                        rows_v[sl] = rows_v[sl] * SCALE

        for g in range(n_chunks + 1):
            if g < n_chunks:
                b = g % 2
                off = base + g * C
                if g >= 2:
                    pltpu.make_async_copy(
                        rows_v.at[b], out_hbm.at[pl.ds(base + (g - 2) * C, C)],
                        wsem[b]).wait()
                pltpu.sync_copy(x_hbm.at[pl.ds(off, C)], idx_v.at[b])
                pltpu.async_copy(table_hbm.at[idx_v.at[b]], rows_v.at[b],
                                 gsem[b])
            if g >= 1:
                p = (g - 1) % 2
                poff = base + (g - 1) * C
                pltpu.make_async_copy(table_hbm.at[idx_v.at[p]],
                                      rows_v.at[p], gsem[p]).wait()
                scale_chunk(p)
                pltpu.async_copy(rows_v.at[p], out_hbm.at[pl.ds(poff, C)],
                                 wsem[p])

        for g in (n_chunks - 2, n_chunks - 1):
            b = g % 2
            pltpu.make_async_copy(
                rows_v.at[b], out_hbm.at[pl.ds(base + g * C, C)],
                wsem[b]).wait()

    return k


def kernel(x, table):
    B = x.shape[0] * x.shape[1]
    x_flat = x.reshape(B)
    out = _make_kernel(B, 800)(x_flat, table)
    return out.reshape(x.shape[0], x.shape[1], D)
